# baseline (speedup 1.0000x reference)
import functools

import jax
import jax.numpy as jnp
import numpy as np
from jax import lax
from jax.experimental import pallas as pl
from jax.experimental.pallas import tpu as pltpu
from jax.experimental.pallas import tpu_sc as plsc

N = 10000
MAXN = 32
DIM = 128
CPC = 4
NCHUNK = N // CPC
NW = 32
NITER = NCHUNK // NW + 1
NBUF = 4


def _proj_body(x_ref, w_ref, b_ref, o_ref):
    o_ref[...] = jax.nn.relu(
        jnp.dot(x_ref[...], w_ref[...], preferred_element_type=jnp.float32)
        + b_ref[...]
    ).astype(jnp.bfloat16)


def _msg_body(x_ref, w_ref, o_ref):
    o_ref[...] = jax.nn.relu(
        jnp.dot(x_ref[...], w_ref[...], preferred_element_type=jnp.float32)
    ).astype(jnp.bfloat16)


def _final_body(a_ref, wmsg_ref, c_ref, wout_ref, bout_ref, logp_ref, h_ref):
    h = jax.nn.relu(
        jnp.dot(a_ref[...], wmsg_ref[...], preferred_element_type=jnp.float32)
    )
    c = c_ref[...]
    hs = jnp.sum(h * h, axis=1, keepdims=True)
    cs = jnp.sum(c * c, axis=1)[None, :]
    sq = hs + cs - 2.0 * jnp.dot(h, c.T, preferred_element_type=jnp.float32)
    sim = jnp.sqrt(jnp.maximum(sq, 1e-12))
    logit = jnp.dot(sim, wout_ref[...], preferred_element_type=jnp.float32)
    logit = logit + bout_ref[...]
    m = jnp.max(logit, axis=1, keepdims=True)
    lse = jnp.log(jnp.sum(jnp.exp(logit - m), axis=1, keepdims=True))
    logp_ref[...] = logit - m - lse
    h_ref[...] = h


_NC = 100
_NCLS = 40
_BLK = 2000


def _tc_proj(x, w, b):
    return pl.pallas_call(
        _proj_body,
        grid=(N // _BLK,),
        in_specs=[
            pl.BlockSpec((_BLK, DIM), lambda i: (i, 0)),
            pl.BlockSpec((DIM, DIM), lambda i: (0, 0)),
            pl.BlockSpec((1, DIM), lambda i: (0, 0)),
        ],
        out_specs=pl.BlockSpec((_BLK, DIM), lambda i: (i, 0)),
        out_shape=jax.ShapeDtypeStruct((N, DIM), jnp.bfloat16),
    )(x, w, b)


def _tc_msg(x, w):
    return pl.pallas_call(
        _msg_body,
        grid=(N // _BLK,),
        in_specs=[
            pl.BlockSpec((_BLK, DIM), lambda i: (i, 0)),
            pl.BlockSpec((DIM, DIM), lambda i: (0, 0)),
        ],
        out_specs=pl.BlockSpec((_BLK, DIM), lambda i: (i, 0)),
        out_shape=jax.ShapeDtypeStruct((N, DIM), jnp.bfloat16),
    )(x, w)


def _tc_final(a, wmsg, centroids, wout, bout):
    return pl.pallas_call(
        _final_body,
        grid=(N // _BLK,),
        in_specs=[
            pl.BlockSpec((_BLK, DIM), lambda i: (i, 0)),
            pl.BlockSpec((DIM, DIM), lambda i: (0, 0)),
            pl.BlockSpec((_NC, DIM), lambda i: (0, 0)),
            pl.BlockSpec((_NC, _NCLS), lambda i: (0, 0)),
            pl.BlockSpec((1, _NCLS), lambda i: (0, 0)),
        ],
        out_specs=[
            pl.BlockSpec((_BLK, _NCLS), lambda i: (i, 0)),
            pl.BlockSpec((_BLK, DIM), lambda i: (i, 0)),
        ],
        out_shape=[
            jax.ShapeDtypeStruct((N, _NCLS), jnp.float32),
            jax.ShapeDtypeStruct((N, DIM), jnp.float32),
        ],
    )(a, wmsg, centroids, wout, bout)


def _agg_compute(rows_v, w_all_v, il, out_v):
    nw = DIM // 32
    for n in range(CPC):
        w0 = w_all_v[il, n, pl.ds(0, 16)]
        w1 = w_all_v[il, n, pl.ds(16, 16)]

        def jstep(j, accs):
            wsel = jnp.where(j < 16, w0, w1)
            lanes = lax.broadcast(j % 16, (16,))
            wb = wsel.at[lanes].get(mode="promise_in_bounds")
            r = n * MAXN + j
            new = []
            for k in range(nw):
                v = rows_v[r, pl.ds(k * 16, 16)]
                lo = lax.bitcast_convert_type(v << 16, jnp.float32)
                hi = lax.bitcast_convert_type(v, jnp.float32)
                new.append(accs[2 * k] + wb * lo)
                new.append(accs[2 * k + 1] + wb * hi)
            return tuple(new)

        accs = lax.fori_loop(
            0, MAXN, jstep,
            tuple(jnp.zeros((16,), jnp.float32) for _ in range(2 * nw)),
            unroll=4,
        )
        for k in range(nw):
            out_v[n, pl.ds(32 * k, 16)] = accs[2 * k]
            out_v[n, pl.ds(32 * k + 16, 16)] = accs[2 * k + 1]


def _agg_body(h_hbm, adj_hbm, w_hbm, out_hbm, idx_all, w_all, rows_v, out_v,
              gat_s, out_s):
    cid = lax.axis_index("c")
    sid = lax.axis_index("s")
    wid = sid * 2 + cid

    pltpu.sync_copy(adj_hbm.at[wid], idx_all)
    pltpu.sync_copy(w_hbm.at[wid], w_all)

    def chunk_of(i):
        return wid + NW * i

    def issue_gather(i, b):
        @pl.when(chunk_of(i) < NCHUNK)
        def _():
            pltpu.async_copy(h_hbm.at[idx_all.at[i]], rows_v[b], gat_s[b])

    for i in range(NBUF - 1):
        issue_gather(i, i)

    def substep(i, b):
        issue_gather(i + NBUF - 1, (b + NBUF - 1) % NBUF)
        c = chunk_of(i)

        @pl.when(c < NCHUNK)
        def _():
            pltpu.make_async_copy(h_hbm.at[idx_all.at[i]], rows_v[b],
                                  gat_s[b]).wait()

            @pl.when(i >= NBUF)
            def _():
                pltpu.make_async_copy(
                    out_v[b], out_hbm.at[pl.ds(c * CPC, CPC)], out_s[b]
                ).wait()

            _agg_compute(rows_v[b], w_all, i, out_v[b])
            pltpu.async_copy(out_v[b], out_hbm.at[pl.ds(c * CPC, CPC)],
                             out_s[b])

    def round_(p, _):
        for u in range(NBUF):
            substep(p * NBUF + u, u)
        return 0

    lax.fori_loop(0, (NITER + NBUF - 1) // NBUF, round_, 0)
    for b in range(NBUF):
        pltpu.make_async_copy(out_v[b], out_hbm.at[pl.ds(0, CPC)],
                              out_s[b]).wait()


@functools.partial(
    pl.kernel,
    out_type=jax.ShapeDtypeStruct((N, DIM), jnp.float32),
    mesh=plsc.VectorSubcoreMesh(
        core_axis_name="c", subcore_axis_name="s",
        num_cores=2, num_subcores=16,
    ),
    compiler_params=pltpu.CompilerParams(use_tc_tiling_on_sc=False),
    scratch_types=[
        pltpu.VMEM((NITER + 1, CPC * MAXN), jnp.int32),
        pltpu.VMEM((NITER + 1, CPC, MAXN), jnp.float32),
        [pltpu.VMEM((CPC * MAXN, DIM // 2), jnp.int32)] * NBUF,
        [pltpu.VMEM((CPC, DIM), jnp.float32)] * NBUF,
        [pltpu.SemaphoreType.DMA] * NBUF,
        [pltpu.SemaphoreType.DMA] * NBUF,
    ],
)
def _sc_agg(h_hbm, adj_hbm, w_hbm, out_hbm, idx_all, w_all, rows_v, out_v,
            gat_s, out_s):
    _agg_body(h_hbm, adj_hbm, w_hbm, out_hbm, idx_all, w_all, rows_v, out_v,
              gat_s, out_s)


_PERM = np.array(
    [32 * k + off
     for k in range(DIM // 32)
     for off in list(range(0, 32, 2)) + list(range(1, 32, 2))],
    dtype=np.int32,
)


def _pack_bf16(h_bf):
    return lax.bitcast_convert_type(
        h_bf.reshape(N, DIM // 2, 2), jnp.int32)


def kernel(adj, weight, features, W_feat, b_feat, W_msg, centroids, W_out,
           b_out):
    npad = NW * (NITER + 1) - NCHUNK
    adj_r = adj[0].astype(jnp.int32).reshape(NCHUNK, CPC * MAXN)
    adj_t = (jnp.pad(adj_r, ((0, npad), (0, 0)))
             .reshape(NITER + 1, NW, CPC * MAXN).transpose(1, 0, 2))
    w_r = weight[0].reshape(NCHUNK, CPC, MAXN)
    w_t = (jnp.pad(w_r, ((0, npad), (0, 0), (0, 0)))
           .reshape(NITER + 1, NW, CPC, MAXN).transpose(1, 0, 2, 3))
    wmsg_p = W_msg[_PERM, :]

    h = _tc_proj(features[0], W_feat, b_feat.reshape(1, DIM))
    a = _sc_agg(_pack_bf16(h), adj_t, w_t)
    h = _tc_msg(a, wmsg_p)
    a = _sc_agg(_pack_bf16(h), adj_t, w_t)
    logp, h2 = _tc_final(a, wmsg_p, centroids, W_out, b_out.reshape(1, _NCLS))
    return logp, h2


# speedup vs baseline: 1.1558x; 1.1558x over previous
import functools

import jax
import jax.numpy as jnp
from jax import lax
from jax.experimental import pallas as pl
from jax.experimental.pallas import tpu as pltpu
from jax.experimental.pallas import tpu_sc as plsc

N = 10000
MAXN = 32
DIM = 128
CPC = 4
NCHUNK = N // CPC
NW = 32
NITER = NCHUNK // NW + 1
NBUF = 4


def _proj_body(x_ref, w_ref, b_ref, o_ref):
    o_ref[...] = jax.nn.relu(
        jnp.dot(x_ref[...], w_ref[...], preferred_element_type=jnp.float32)
        + b_ref[...]
    )


def _msg_body(x_ref, w_ref, o_ref):
    o_ref[...] = jax.nn.relu(
        jnp.dot(x_ref[...], w_ref[...], preferred_element_type=jnp.float32)
    )


def _final_body(a_ref, wmsg_ref, c_ref, wout_ref, bout_ref, logp_ref, h_ref):
    h = jax.nn.relu(
        jnp.dot(a_ref[...], wmsg_ref[...], preferred_element_type=jnp.float32)
    )
    c = c_ref[...]
    hs = jnp.sum(h * h, axis=1, keepdims=True)
    cs = jnp.sum(c * c, axis=1)[None, :]
    sq = hs + cs - 2.0 * jnp.dot(h, c.T, preferred_element_type=jnp.float32)
    sim = jnp.sqrt(jnp.maximum(sq, 1e-12))
    logit = jnp.dot(sim, wout_ref[...], preferred_element_type=jnp.float32)
    logit = logit + bout_ref[...]
    m = jnp.max(logit, axis=1, keepdims=True)
    lse = jnp.log(jnp.sum(jnp.exp(logit - m), axis=1, keepdims=True))
    logp_ref[...] = logit - m - lse
    h_ref[...] = h


_NC = 100
_NCLS = 40
_BLK = 2000


def _tc_proj(x, w, b):
    return pl.pallas_call(
        _proj_body,
        grid=(N // _BLK,),
        in_specs=[
            pl.BlockSpec((_BLK, DIM), lambda i: (i, 0)),
            pl.BlockSpec((DIM, DIM), lambda i: (0, 0)),
            pl.BlockSpec((1, DIM), lambda i: (0, 0)),
        ],
        out_specs=pl.BlockSpec((_BLK, DIM), lambda i: (i, 0)),
        out_shape=jax.ShapeDtypeStruct((N, DIM), jnp.float32),
    )(x, w, b)


def _tc_msg(x, w):
    return pl.pallas_call(
        _msg_body,
        grid=(N // _BLK,),
        in_specs=[
            pl.BlockSpec((_BLK, DIM), lambda i: (i, 0)),
            pl.BlockSpec((DIM, DIM), lambda i: (0, 0)),
        ],
        out_specs=pl.BlockSpec((_BLK, DIM), lambda i: (i, 0)),
        out_shape=jax.ShapeDtypeStruct((N, DIM), jnp.float32),
    )(x, w)


def _tc_final(a, wmsg, centroids, wout, bout):
    return pl.pallas_call(
        _final_body,
        grid=(N // _BLK,),
        in_specs=[
            pl.BlockSpec((_BLK, DIM), lambda i: (i, 0)),
            pl.BlockSpec((DIM, DIM), lambda i: (0, 0)),
            pl.BlockSpec((_NC, DIM), lambda i: (0, 0)),
            pl.BlockSpec((_NC, _NCLS), lambda i: (0, 0)),
            pl.BlockSpec((1, _NCLS), lambda i: (0, 0)),
        ],
        out_specs=[
            pl.BlockSpec((_BLK, _NCLS), lambda i: (i, 0)),
            pl.BlockSpec((_BLK, DIM), lambda i: (i, 0)),
        ],
        out_shape=[
            jax.ShapeDtypeStruct((N, _NCLS), jnp.float32),
            jax.ShapeDtypeStruct((N, DIM), jnp.float32),
        ],
    )(a, wmsg, centroids, wout, bout)


def _agg_compute(rows_v, w_all_v, il, out_v):
    nk = DIM // 16
    for n in range(CPC):
        w0 = w_all_v[il, n, pl.ds(0, 16)]
        w1 = w_all_v[il, n, pl.ds(16, 16)]

        def jstep(j, accs):
            wsel = jnp.where(j < 16, w0, w1)
            lanes = lax.broadcast(j % 16, (16,))
            wb = wsel.at[lanes].get(mode="promise_in_bounds")
            r = n * MAXN + j
            return tuple(
                accs[k] + wb * rows_v[r, pl.ds(k * 16, 16)]
                for k in range(nk)
            )

        accs = lax.fori_loop(
            0, MAXN, jstep,
            tuple(jnp.zeros((16,), jnp.float32) for _ in range(nk)),
            unroll=4,
        )
        for k in range(nk):
            out_v[n, pl.ds(k * 16, 16)] = accs[k]


def _agg_body(h_hbm, adj_hbm, w_hbm, out_hbm, idx_all, w_all, rows_v, out_v,
              gat_s, out_s):
    cid = lax.axis_index("c")
    sid = lax.axis_index("s")
    wid = sid * 2 + cid

    pltpu.sync_copy(adj_hbm.at[wid], idx_all)
    pltpu.sync_copy(w_hbm.at[wid], w_all)

    def chunk_of(i):
        return wid + NW * i

    def issue_gather(i, b):
        @pl.when(chunk_of(i) < NCHUNK)
        def _():
            pltpu.async_copy(h_hbm.at[idx_all.at[i]], rows_v[b], gat_s[b])

    for i in range(NBUF - 1):
        issue_gather(i, i)

    def substep(i, b):
        issue_gather(i + NBUF - 1, (b + NBUF - 1) % NBUF)
        c = chunk_of(i)

        @pl.when(c < NCHUNK)
        def _():
            pltpu.make_async_copy(h_hbm.at[idx_all.at[i]], rows_v[b],
                                  gat_s[b]).wait()

            @pl.when(i >= NBUF)
            def _():
                pltpu.make_async_copy(
                    out_v[b], out_hbm.at[pl.ds(c * CPC, CPC)], out_s[b]
                ).wait()

            _agg_compute(rows_v[b], w_all, i, out_v[b])
            pltpu.async_copy(out_v[b], out_hbm.at[pl.ds(c * CPC, CPC)],
                             out_s[b])

    def round_(p, _):
        for u in range(NBUF):
            substep(p * NBUF + u, u)
        return 0

    lax.fori_loop(0, (NITER + NBUF - 1) // NBUF, round_, 0)
    for b in range(NBUF):
        pltpu.make_async_copy(out_v[b], out_hbm.at[pl.ds(0, CPC)],
                              out_s[b]).wait()


@functools.partial(
    pl.kernel,
    out_type=jax.ShapeDtypeStruct((N, DIM), jnp.float32),
    mesh=plsc.VectorSubcoreMesh(
        core_axis_name="c", subcore_axis_name="s",
        num_cores=2, num_subcores=16,
    ),
    scratch_types=[
        pltpu.VMEM((NITER + 1, CPC * MAXN), jnp.int32),
        pltpu.VMEM((NITER + 1, CPC, MAXN), jnp.float32),
        [pltpu.VMEM((CPC * MAXN, DIM), jnp.float32)] * NBUF,
        [pltpu.VMEM((CPC, DIM), jnp.float32)] * NBUF,
        [pltpu.SemaphoreType.DMA] * NBUF,
        [pltpu.SemaphoreType.DMA] * NBUF,
    ],
)
def _sc_agg(h_hbm, adj_hbm, w_hbm, out_hbm, idx_all, w_all, rows_v, out_v,
            gat_s, out_s):
    _agg_body(h_hbm, adj_hbm, w_hbm, out_hbm, idx_all, w_all, rows_v, out_v,
              gat_s, out_s)


def kernel(adj, weight, features, W_feat, b_feat, W_msg, centroids, W_out,
           b_out):
    npad = NW * (NITER + 1) - NCHUNK
    adj_r = adj[0].astype(jnp.int32).reshape(NCHUNK, CPC * MAXN)
    adj_t = (jnp.pad(adj_r, ((0, npad), (0, 0)))
             .reshape(NITER + 1, NW, CPC * MAXN).transpose(1, 0, 2))
    w_r = weight[0].reshape(NCHUNK, CPC, MAXN)
    w_t = (jnp.pad(w_r, ((0, npad), (0, 0), (0, 0)))
           .reshape(NITER + 1, NW, CPC, MAXN).transpose(1, 0, 2, 3))

    h = _tc_proj(features[0], W_feat, b_feat.reshape(1, DIM))
    a = _sc_agg(h, adj_t, w_t)
    h = _tc_msg(a, W_msg)
    a = _sc_agg(h, adj_t, w_t)
    logp, h2 = _tc_final(a, W_msg, centroids, W_out, b_out.reshape(1, _NCLS))
    return logp, h2
